# initial kernel scaffold (unmeasured)
import jax
import jax.numpy as jnp
from jax import lax
from jax.experimental import pallas as pl
from jax.experimental.pallas import tpu as pltpu

N_DEV = 8


def kernel(x, w_mat):
    m_total, k_blk = x.shape
    k_total, n = w_mat.shape
    m_blk = m_total // N_DEV

    def body(x_hbm, w_ref, out_ref, recv_buf, send_sems, recv_sems, local_sem):
        j = pl.program_id(0)
        my = lax.axis_index("i")

        @pl.when(j == 0)
        def _():
            pltpu.make_async_copy(
                x_hbm.at[pl.ds(my * m_blk, m_blk), :],
                recv_buf.at[my],
                local_sem,
            ).start()
            barrier = pltpu.get_barrier_semaphore()
            for o in range(1, N_DEV):
                d = lax.rem(my + o, N_DEV)
                pl.semaphore_signal(
                    barrier, inc=1,
                    device_id=(d,), device_id_type=pl.DeviceIdType.MESH,
                )
            pl.semaphore_wait(barrier, N_DEV - 1)
            for o in range(1, N_DEV):
                d = lax.rem(my - o + N_DEV, N_DEV)
                pltpu.make_async_remote_copy(
                    src_ref=x_hbm.at[pl.ds(d * m_blk, m_blk), :],
                    dst_ref=recv_buf.at[my],
                    send_sem=send_sems.at[o],
                    recv_sem=recv_sems.at[my],
                    device_id=(d,),
                    device_id_type=pl.DeviceIdType.MESH,
                ).start()

        @pl.when(j == my)
        def _():
            pltpu.make_async_copy(
                x_hbm.at[pl.ds(my * m_blk, m_blk), :],
                recv_buf.at[my],
                local_sem,
            ).wait()

        @pl.when(j != my)
        def _():
            pltpu.make_async_remote_copy(
                src_ref=x_hbm.at[pl.ds(0, m_blk), :],
                dst_ref=recv_buf.at[j],
                send_sem=send_sems.at[0],
                recv_sem=recv_sems.at[j],
                device_id=(my,),
                device_id_type=pl.DeviceIdType.MESH,
            ).wait_recv()

        res = jnp.dot(recv_buf[j], w_ref[...], preferred_element_type=jnp.float32)

        @pl.when(j == 0)
        def _():
            out_ref[...] = res

        @pl.when(jnp.logical_and(j > 0, j < N_DEV - 1))
        def _():
            out_ref[...] += res

        @pl.when(j == N_DEV - 1)
        def _():
            out_ref[...] = jnp.maximum(out_ref[...] + res, 0.0)
            for o in range(1, N_DEV):
                d = lax.rem(my - o + N_DEV, N_DEV)
                pltpu.make_async_remote_copy(
                    src_ref=x_hbm.at[pl.ds(d * m_blk, m_blk), :],
                    dst_ref=recv_buf.at[my],
                    send_sem=send_sems.at[o],
                    recv_sem=recv_sems.at[my],
                    device_id=(d,),
                    device_id_type=pl.DeviceIdType.MESH,
                ).wait_send()

    return pl.pallas_call(
        body,
        grid=(N_DEV,),
        out_shape=jax.ShapeDtypeStruct((m_blk, n), jnp.float32),
        in_specs=[
            pl.BlockSpec(memory_space=pltpu.ANY),
            pl.BlockSpec((k_total // N_DEV, n), lambda j: (j, 0)),
        ],
        out_specs=pl.BlockSpec((m_blk, n), lambda j: (0, 0)),
        scratch_shapes=[
            pltpu.VMEM((N_DEV, m_blk, k_blk), jnp.float32),
            pltpu.SemaphoreType.DMA((N_DEV,)),
            pltpu.SemaphoreType.DMA((N_DEV,)),
            pltpu.SemaphoreType.DMA,
        ],
        compiler_params=pltpu.CompilerParams(
            collective_id=0,
            dimension_semantics=("arbitrary",),
            vmem_limit_bytes=128 * 1024 * 1024,
        ),
    )(x, w_mat)


# baseline (device time: 380214 ns/iter reference)
import jax
import jax.numpy as jnp
from jax import lax
from jax.experimental import pallas as pl
from jax.experimental.pallas import tpu as pltpu

N_DEV = 8
N_CHUNKS = 4


def kernel(x, w_mat):
    m_total, k_blk = x.shape
    k_total, n = w_mat.shape
    m_blk = m_total // N_DEV
    n_blk = n // N_CHUNKS

    def body(x_hbm, w_ref, out_ref, recv_buf, send_sems, recv_sems, local_sem):
        nn = pl.program_id(0)
        j = pl.program_id(1)
        my = lax.axis_index("i")
        first_step = jnp.logical_and(nn == 0, j == 0)
        last_step = jnp.logical_and(nn == N_CHUNKS - 1, j == N_DEV - 1)

        @pl.when(first_step)
        def _():
            pltpu.make_async_copy(
                x_hbm.at[pl.ds(my * m_blk, m_blk), :],
                recv_buf.at[my],
                local_sem,
            ).start()
            barrier = pltpu.get_barrier_semaphore()
            for o in range(1, N_DEV):
                d = lax.rem(my + o, N_DEV)
                pl.semaphore_signal(
                    barrier, inc=1,
                    device_id=(d,), device_id_type=pl.DeviceIdType.MESH,
                )
            pl.semaphore_wait(barrier, N_DEV - 1)
            for o in range(1, N_DEV):
                d = lax.rem(my - o + N_DEV, N_DEV)
                pltpu.make_async_remote_copy(
                    src_ref=x_hbm.at[pl.ds(d * m_blk, m_blk), :],
                    dst_ref=recv_buf.at[my],
                    send_sem=send_sems.at[o],
                    recv_sem=recv_sems.at[my],
                    device_id=(d,),
                    device_id_type=pl.DeviceIdType.MESH,
                ).start()

        @pl.when(jnp.logical_and(nn == 0, j == my))
        def _():
            pltpu.make_async_copy(
                x_hbm.at[pl.ds(my * m_blk, m_blk), :],
                recv_buf.at[my],
                local_sem,
            ).wait()

        @pl.when(jnp.logical_and(nn == 0, j != my))
        def _():
            pltpu.make_async_remote_copy(
                src_ref=x_hbm.at[pl.ds(0, m_blk), :],
                dst_ref=recv_buf.at[j],
                send_sem=send_sems.at[0],
                recv_sem=recv_sems.at[j],
                device_id=(my,),
                device_id_type=pl.DeviceIdType.MESH,
            ).wait_recv()

        res = jnp.dot(recv_buf[j], w_ref[...], preferred_element_type=jnp.float32)

        @pl.when(j == 0)
        def _():
            out_ref[...] = res

        @pl.when(jnp.logical_and(j > 0, j < N_DEV - 1))
        def _():
            out_ref[...] += res

        @pl.when(j == N_DEV - 1)
        def _():
            out_ref[...] = jnp.maximum(out_ref[...] + res, 0.0)

        @pl.when(last_step)
        def _():
            for o in range(1, N_DEV):
                d = lax.rem(my - o + N_DEV, N_DEV)
                pltpu.make_async_remote_copy(
                    src_ref=x_hbm.at[pl.ds(d * m_blk, m_blk), :],
                    dst_ref=recv_buf.at[my],
                    send_sem=send_sems.at[o],
                    recv_sem=recv_sems.at[my],
                    device_id=(d,),
                    device_id_type=pl.DeviceIdType.MESH,
                ).wait_send()

    return pl.pallas_call(
        body,
        grid=(N_CHUNKS, N_DEV),
        out_shape=jax.ShapeDtypeStruct((m_blk, n), jnp.float32),
        in_specs=[
            pl.BlockSpec(memory_space=pl.ANY),
            pl.BlockSpec((k_total // N_DEV, n_blk), lambda nn, j: (j, nn)),
        ],
        out_specs=pl.BlockSpec((m_blk, n_blk), lambda nn, j: (0, nn)),
        scratch_shapes=[
            pltpu.VMEM((N_DEV, m_blk, k_blk), jnp.float32),
            pltpu.SemaphoreType.DMA((N_DEV,)),
            pltpu.SemaphoreType.DMA((N_DEV,)),
            pltpu.SemaphoreType.DMA,
        ],
        compiler_params=pltpu.CompilerParams(
            collective_id=0,
            dimension_semantics=("arbitrary", "arbitrary"),
            vmem_limit_bytes=64 * 1024 * 1024,
        ),
    )(x, w_mat)


# device time: 370455 ns/iter; 1.0263x vs baseline; 1.0263x over previous
import jax
import jax.numpy as jnp
from jax import lax
from jax.experimental import pallas as pl
from jax.experimental.pallas import tpu as pltpu

N_DEV = 8
N_CHUNKS = 4


def kernel(x, w_mat):
    m_total, k_blk = x.shape
    k_total, n = w_mat.shape
    m_blk = m_total // N_DEV
    n_blk = n // N_CHUNKS

    def body(perm_ref, x_hbm, w_ref, out_ref,
             recv_buf, send_sems, recv_sems, local_sem):
        nn = pl.program_id(0)
        j = pl.program_id(1)
        my = lax.axis_index("i")
        k = perm_ref[j]
        first_step = jnp.logical_and(nn == 0, j == 0)
        last_step = jnp.logical_and(nn == N_CHUNKS - 1, j == N_DEV - 1)

        @pl.when(first_step)
        def _():
            pltpu.make_async_copy(
                x_hbm.at[pl.ds(my * m_blk, m_blk), :],
                recv_buf.at[my],
                local_sem,
            ).start()
            barrier = pltpu.get_barrier_semaphore()
            for o in range(1, N_DEV):
                d = lax.rem(my + o, N_DEV)
                pl.semaphore_signal(
                    barrier, inc=1,
                    device_id=(d,), device_id_type=pl.DeviceIdType.MESH,
                )
            pl.semaphore_wait(barrier, N_DEV - 1)
            for o in range(1, N_DEV):
                d = lax.rem(my - o + N_DEV, N_DEV)
                pltpu.make_async_remote_copy(
                    src_ref=x_hbm.at[pl.ds(d * m_blk, m_blk), :],
                    dst_ref=recv_buf.at[my],
                    send_sem=send_sems.at[o],
                    recv_sem=recv_sems.at[my],
                    device_id=(d,),
                    device_id_type=pl.DeviceIdType.MESH,
                ).start()

        @pl.when(jnp.logical_and(nn == 0, j == 0))
        def _():
            pltpu.make_async_copy(
                x_hbm.at[pl.ds(my * m_blk, m_blk), :],
                recv_buf.at[my],
                local_sem,
            ).wait()

        @pl.when(jnp.logical_and(nn == 0, j != 0))
        def _():
            pltpu.make_async_remote_copy(
                src_ref=x_hbm.at[pl.ds(0, m_blk), :],
                dst_ref=recv_buf.at[k],
                send_sem=send_sems.at[0],
                recv_sem=recv_sems.at[k],
                device_id=(my,),
                device_id_type=pl.DeviceIdType.MESH,
            ).wait_recv()

        res = jnp.dot(recv_buf[k], w_ref[...], preferred_element_type=jnp.float32)

        @pl.when(j == 0)
        def _():
            out_ref[...] = res

        @pl.when(jnp.logical_and(j > 0, j < N_DEV - 1))
        def _():
            out_ref[...] += res

        @pl.when(j == N_DEV - 1)
        def _():
            out_ref[...] = jnp.maximum(out_ref[...] + res, 0.0)

        @pl.when(last_step)
        def _():
            for o in range(1, N_DEV):
                d = lax.rem(my - o + N_DEV, N_DEV)
                pltpu.make_async_remote_copy(
                    src_ref=x_hbm.at[pl.ds(d * m_blk, m_blk), :],
                    dst_ref=recv_buf.at[my],
                    send_sem=send_sems.at[o],
                    recv_sem=recv_sems.at[my],
                    device_id=(d,),
                    device_id_type=pl.DeviceIdType.MESH,
                ).wait_send()

    my_pos = lax.axis_index("i")
    perm = lax.rem(my_pos + jnp.arange(N_DEV, dtype=jnp.int32), N_DEV)

    grid_spec = pltpu.PrefetchScalarGridSpec(
        num_scalar_prefetch=1,
        grid=(N_CHUNKS, N_DEV),
        in_specs=[
            pl.BlockSpec(memory_space=pl.ANY),
            pl.BlockSpec(
                (k_total // N_DEV, n_blk),
                lambda nn, j, perm_ref: (perm_ref[j], nn),
            ),
        ],
        out_specs=pl.BlockSpec((m_blk, n_blk), lambda nn, j, perm_ref: (0, nn)),
        scratch_shapes=[
            pltpu.VMEM((N_DEV, m_blk, k_blk), jnp.float32),
            pltpu.SemaphoreType.DMA((N_DEV,)),
            pltpu.SemaphoreType.DMA((N_DEV,)),
            pltpu.SemaphoreType.DMA,
        ],
    )

    return pl.pallas_call(
        body,
        grid_spec=grid_spec,
        out_shape=jax.ShapeDtypeStruct((m_blk, n), jnp.float32),
        compiler_params=pltpu.CompilerParams(
            collective_id=0,
            dimension_semantics=("arbitrary", "arbitrary"),
            vmem_limit_bytes=64 * 1024 * 1024,
        ),
    )(perm, x, w_mat)


# device time: 202449 ns/iter; 1.8781x vs baseline; 1.8299x over previous
import jax
import jax.numpy as jnp
from jax import lax
from jax.experimental import pallas as pl
from jax.experimental.pallas import tpu as pltpu

N_DEV = 8
N_CHUNKS = 4


def kernel(x, w_mat):
    m_total, k_blk = x.shape
    k_total, n = w_mat.shape
    m_blk = m_total // N_DEV
    n_blk = n // N_CHUNKS

    x16 = x.astype(jnp.bfloat16)

    def body(perm_ref, x_hbm, w_ref, out_ref,
             recv_buf, acc_ref, send_sems, recv_sems, local_sem):
        j = pl.program_id(0)
        nn = pl.program_id(1)
        my = lax.axis_index("i")
        k = perm_ref[j]
        first_step = jnp.logical_and(j == 0, nn == 0)
        last_step = jnp.logical_and(j == N_DEV - 1, nn == N_CHUNKS - 1)

        @pl.when(first_step)
        def _():
            pltpu.make_async_copy(
                x_hbm.at[pl.ds(my * m_blk, m_blk), :],
                recv_buf.at[my],
                local_sem,
            ).start()
            barrier = pltpu.get_barrier_semaphore()
            for o in range(1, N_DEV):
                d = lax.rem(my + o, N_DEV)
                pl.semaphore_signal(
                    barrier, inc=1,
                    device_id=(d,), device_id_type=pl.DeviceIdType.MESH,
                )
            pl.semaphore_wait(barrier, N_DEV - 1)
            for o in range(1, N_DEV):
                d = lax.rem(my - o + N_DEV, N_DEV)
                pltpu.make_async_remote_copy(
                    src_ref=x_hbm.at[pl.ds(d * m_blk, m_blk), :],
                    dst_ref=recv_buf.at[my],
                    send_sem=send_sems.at[o],
                    recv_sem=recv_sems.at[my],
                    device_id=(d,),
                    device_id_type=pl.DeviceIdType.MESH,
                ).start()

        @pl.when(jnp.logical_and(nn == 0, j == 0))
        def _():
            pltpu.make_async_copy(
                x_hbm.at[pl.ds(my * m_blk, m_blk), :],
                recv_buf.at[my],
                local_sem,
            ).wait()

        @pl.when(jnp.logical_and(nn == 0, j != 0))
        def _():
            pltpu.make_async_remote_copy(
                src_ref=x_hbm.at[pl.ds(0, m_blk), :],
                dst_ref=recv_buf.at[k],
                send_sem=send_sems.at[0],
                recv_sem=recv_sems.at[k],
                device_id=(my,),
                device_id_type=pl.DeviceIdType.MESH,
            ).wait_recv()

        res = jnp.dot(
            recv_buf[k].astype(jnp.float32), w_ref[...],
            preferred_element_type=jnp.float32,
        )

        @pl.when(j == 0)
        def _():
            acc_ref[nn] = res

        @pl.when(jnp.logical_and(j > 0, j < N_DEV - 1))
        def _():
            acc_ref[nn] += res

        @pl.when(j == N_DEV - 1)
        def _():
            acc_ref[nn] = jnp.maximum(acc_ref[nn] + res, 0.0)

        @pl.when(last_step)
        def _():
            for c in range(N_CHUNKS):
                out_ref[:, c * n_blk:(c + 1) * n_blk] = acc_ref[c]
            for o in range(1, N_DEV):
                d = lax.rem(my - o + N_DEV, N_DEV)
                pltpu.make_async_remote_copy(
                    src_ref=x_hbm.at[pl.ds(d * m_blk, m_blk), :],
                    dst_ref=recv_buf.at[my],
                    send_sem=send_sems.at[o],
                    recv_sem=recv_sems.at[my],
                    device_id=(d,),
                    device_id_type=pl.DeviceIdType.MESH,
                ).wait_send()

    my_pos = lax.axis_index("i")
    perm = lax.rem(my_pos + jnp.arange(N_DEV, dtype=jnp.int32), N_DEV)

    grid_spec = pltpu.PrefetchScalarGridSpec(
        num_scalar_prefetch=1,
        grid=(N_DEV, N_CHUNKS),
        in_specs=[
            pl.BlockSpec(memory_space=pl.ANY),
            pl.BlockSpec(
                (k_total // N_DEV, n_blk),
                lambda j, nn, perm_ref: (perm_ref[j], nn),
            ),
        ],
        out_specs=pl.BlockSpec((m_blk, n), lambda j, nn, perm_ref: (0, 0)),
        scratch_shapes=[
            pltpu.VMEM((N_DEV, m_blk, k_blk), jnp.bfloat16),
            pltpu.VMEM((N_CHUNKS, m_blk, n_blk), jnp.float32),
            pltpu.SemaphoreType.DMA((N_DEV,)),
            pltpu.SemaphoreType.DMA((N_DEV,)),
            pltpu.SemaphoreType.DMA,
        ],
    )

    return pl.pallas_call(
        body,
        grid_spec=grid_spec,
        out_shape=jax.ShapeDtypeStruct((m_blk, n), jnp.float32),
        compiler_params=pltpu.CompilerParams(
            collective_id=0,
            dimension_semantics=("arbitrary", "arbitrary"),
            vmem_limit_bytes=64 * 1024 * 1024,
        ),
    )(perm, x16, w_mat)


# device time: 194387 ns/iter; 1.9560x vs baseline; 1.0415x over previous
import jax
import jax.numpy as jnp
from jax import lax
from jax.experimental import pallas as pl
from jax.experimental.pallas import tpu as pltpu

N_DEV = 8
N_CHUNKS = 4


def kernel(x, w_mat):
    m_total, k_blk = x.shape
    k_total, n = w_mat.shape
    m_blk = m_total // N_DEV
    n_blk = n // N_CHUNKS

    x16 = x.astype(jnp.bfloat16)

    def body(perm_ref, x_hbm, w_ref, out_ref,
             recv_buf, acc_ref, send_sems, recv_sems, local_sem):
        j = pl.program_id(0)
        nn = pl.program_id(1)
        my = lax.axis_index("i")
        k = perm_ref[j]
        first_step = jnp.logical_and(j == 0, nn == 0)
        last_step = jnp.logical_and(j == N_DEV - 1, nn == N_CHUNKS - 1)

        @pl.when(first_step)
        def _():
            pltpu.make_async_copy(
                x_hbm.at[pl.ds(my * m_blk, m_blk), :],
                recv_buf.at[my],
                local_sem,
            ).start()
            barrier = pltpu.get_barrier_semaphore()
            for o in range(1, N_DEV):
                d = lax.rem(my + o, N_DEV)
                pl.semaphore_signal(
                    barrier, inc=1,
                    device_id=(d,), device_id_type=pl.DeviceIdType.MESH,
                )
            pl.semaphore_wait(barrier, N_DEV - 1)

        for o in range(1, N_DEV):
            issue_j = max(0, o - 2)

            @pl.when(jnp.logical_and(nn == 0, j == issue_j))
            def _(o=o):
                d = lax.rem(my - o + N_DEV, N_DEV)
                pltpu.make_async_remote_copy(
                    src_ref=x_hbm.at[pl.ds(d * m_blk, m_blk), :],
                    dst_ref=recv_buf.at[my],
                    send_sem=send_sems.at[o],
                    recv_sem=recv_sems.at[my],
                    device_id=(d,),
                    device_id_type=pl.DeviceIdType.MESH,
                ).start()

        @pl.when(jnp.logical_and(nn == 0, j == 0))
        def _():
            pltpu.make_async_copy(
                x_hbm.at[pl.ds(my * m_blk, m_blk), :],
                recv_buf.at[my],
                local_sem,
            ).wait()

        @pl.when(jnp.logical_and(nn == 0, j != 0))
        def _():
            pltpu.make_async_remote_copy(
                src_ref=x_hbm.at[pl.ds(0, m_blk), :],
                dst_ref=recv_buf.at[k],
                send_sem=send_sems.at[0],
                recv_sem=recv_sems.at[k],
                device_id=(my,),
                device_id_type=pl.DeviceIdType.MESH,
            ).wait_recv()

        res = jnp.dot(
            recv_buf[k].astype(jnp.float32), w_ref[...],
            preferred_element_type=jnp.float32,
        )

        @pl.when(j == 0)
        def _():
            acc_ref[nn] = res

        @pl.when(jnp.logical_and(j > 0, j < N_DEV - 1))
        def _():
            acc_ref[nn] += res

        @pl.when(j == N_DEV - 1)
        def _():
            acc_ref[nn] = jnp.maximum(acc_ref[nn] + res, 0.0)

        @pl.when(last_step)
        def _():
            for c in range(N_CHUNKS):
                out_ref[:, c * n_blk:(c + 1) * n_blk] = acc_ref[c]
            for o in range(1, N_DEV):
                d = lax.rem(my - o + N_DEV, N_DEV)
                pltpu.make_async_remote_copy(
                    src_ref=x_hbm.at[pl.ds(d * m_blk, m_blk), :],
                    dst_ref=recv_buf.at[my],
                    send_sem=send_sems.at[o],
                    recv_sem=recv_sems.at[my],
                    device_id=(d,),
                    device_id_type=pl.DeviceIdType.MESH,
                ).wait_send()

    my_pos = lax.axis_index("i")
    perm = lax.rem(my_pos + jnp.arange(N_DEV, dtype=jnp.int32), N_DEV)

    grid_spec = pltpu.PrefetchScalarGridSpec(
        num_scalar_prefetch=1,
        grid=(N_DEV, N_CHUNKS),
        in_specs=[
            pl.BlockSpec(memory_space=pl.ANY),
            pl.BlockSpec(
                (k_total // N_DEV, n_blk),
                lambda j, nn, perm_ref: (perm_ref[j], nn),
            ),
        ],
        out_specs=pl.BlockSpec((m_blk, n), lambda j, nn, perm_ref: (0, 0)),
        scratch_shapes=[
            pltpu.VMEM((N_DEV, m_blk, k_blk), jnp.bfloat16),
            pltpu.VMEM((N_CHUNKS, m_blk, n_blk), jnp.float32),
            pltpu.SemaphoreType.DMA((N_DEV,)),
            pltpu.SemaphoreType.DMA((N_DEV,)),
            pltpu.SemaphoreType.DMA,
        ],
    )

    return pl.pallas_call(
        body,
        grid_spec=grid_spec,
        out_shape=jax.ShapeDtypeStruct((m_blk, n), jnp.float32),
        compiler_params=pltpu.CompilerParams(
            collective_id=0,
            dimension_semantics=("arbitrary", "arbitrary"),
            vmem_limit_bytes=64 * 1024 * 1024,
        ),
    )(perm, x16, w_mat)
